# baseline (speedup 1.0000x reference)
import functools

import jax
import jax.numpy as jnp
from jax import lax
from jax.experimental import pallas as pl
from jax.experimental.pallas import tpu as pltpu
from jax.experimental.pallas import tpu_sc as plsc

_NW = 32
_NPW = 320
_NPAD = _NW * _NPW
_SINK = _NPW
_LCAP = 43 * 4096


def _sc_mesh():
    return plsc.VectorSubcoreMesh(core_axis_name="c", subcore_axis_name="s")


def _build_lists(dst):
    e_total = dst.shape[0]
    assert e_total % 4000 == 0
    nblocks = e_total // 4000

    @functools.partial(
        pl.kernel, mesh=_sc_mesh(),
        compiler_params=pltpu.CompilerParams(needs_layout_passes=False),
        out_type=[
            jax.ShapeDtypeStruct((_NW * _LCAP,), jnp.int32),
            jax.ShapeDtypeStruct((_NW * _LCAP,), jnp.int32),
            jax.ShapeDtypeStruct((_NW * 16,), jnp.int32),
        ],
        scratch_types=[
            pltpu.VMEM((4000,), jnp.int32),
            pltpu.VMEM((8224,), jnp.int32),
            pltpu.VMEM((8224,), jnp.int32),
            pltpu.VMEM((16,), jnp.int32),
            pltpu.SemaphoreType.DMA,
        ],
    )
    def k(dst_hbm, le_hbm, lr_hbm, cnt_hbm, dbuf, be, br, cv, sem):
        wid = lax.axis_index("s") * 2 + lax.axis_index("c")
        lbase = wid * _LCAP
        lo = wid * _NPW
        lane = lax.iota(jnp.int32, 16)

        def block(b, carry):
            cnt, hcnt = carry
            pltpu.sync_copy(dst_hbm.at[pl.ds(pl.multiple_of(b * 4000, 8), 4000)], dbuf)

            def scan16(k2, cnt):
                d = dbuf[pl.ds(k2 * 16, 16)]
                m = (d >= lo) & (d < lo + _NPW)
                eid = b * 4000 + k2 * 16 + lane
                mi = m.astype(jnp.int32)
                pos = plsc.cumsum(mi)
                tgt = cnt + pos - mi
                plsc.store_scatter(be, [tgt], eid, mask=m)
                plsc.store_scatter(br, [tgt], (d - lo) * 128, mask=m)
                return cnt + pos[15]

            cnt = lax.fori_loop(0, 250, scan16, cnt)

            def flush(args):
                cnt, hcnt = args
                pltpu.sync_copy(be.at[pl.ds(0, 4096)],
                                le_hbm.at[pl.ds(pl.multiple_of(lbase + hcnt, 8), 4096)])
                pltpu.sync_copy(br.at[pl.ds(0, 4096)],
                                lr_hbm.at[pl.ds(pl.multiple_of(lbase + hcnt, 8), 4096)])

                def mv(i, _):
                    be[pl.ds(i * 16, 16)] = be[pl.ds(4096 + i * 16, 16)]
                    br[pl.ds(i * 16, 16)] = br[pl.ds(4096 + i * 16, 16)]
                    return 0

                lax.fori_loop(0, 256, mv, 0)
                return cnt - 4096, hcnt + 4096

            return lax.cond(cnt >= 4096, flush, lambda a: a, (cnt, hcnt))

        cnt, hcnt = lax.fori_loop(0, nblocks, block,
                                  (jnp.int32(0), jnp.int32(0)))
        for i in range(8):
            be[pl.ds(cnt + i * 16, 16)] = jnp.zeros((16,), jnp.int32)
            br[pl.ds(cnt + i * 16, 16)] = jnp.full((16,), _SINK * 128, jnp.int32)
        cntp = ((cnt + 127) >> 7) << 7

        def fflush(i, _):
            pltpu.sync_copy(be.at[pl.ds(i * 4096, 4096)],
                            le_hbm.at[pl.ds(pl.multiple_of(lbase + hcnt + i * 4096, 8), 4096)])
            pltpu.sync_copy(br.at[pl.ds(i * 4096, 4096)],
                            lr_hbm.at[pl.ds(pl.multiple_of(lbase + hcnt + i * 4096, 8), 4096)])
            return 0

        lax.fori_loop(0, (cntp + 4095) >> 12, fflush, 0)
        cv[...] = jnp.full((16,), hcnt + cntp, jnp.int32)
        pltpu.sync_copy(cv, cnt_hbm.at[pl.ds(pl.multiple_of(wid * 16, 8), 16)])

    return k(dst)


_GC = 64


def _gather_add(pq, dst, src):
    e_total = dst.shape[0]
    assert e_total % _GC == 0
    nch_total = e_total // _GC

    @functools.partial(
        pl.kernel, mesh=_sc_mesh(),
        compiler_params=pltpu.CompilerParams(needs_layout_passes=False),
        out_type=jax.ShapeDtypeStruct((e_total, 256), jnp.float32),
        scratch_types=[
            pltpu.VMEM((_GC,), jnp.int32),
            pltpu.VMEM((_GC,), jnp.int32),
            pltpu.VMEM((_GC,), jnp.int32),
            pltpu.VMEM((_GC,), jnp.int32),
            pltpu.VMEM((_GC,), jnp.int32),
            pltpu.VMEM((_GC,), jnp.int32),
            pltpu.VMEM((_GC,), jnp.int32),
            pltpu.VMEM((_GC,), jnp.int32),
            pltpu.VMEM((_GC, 256), jnp.float32),
            pltpu.VMEM((_GC, 256), jnp.float32),
            pltpu.VMEM((_GC, 256), jnp.float32),
            pltpu.VMEM((_GC, 256), jnp.float32),
            pltpu.SemaphoreType.DMA,
            pltpu.SemaphoreType.DMA,
            pltpu.SemaphoreType.DMA,
            pltpu.SemaphoreType.DMA,
            pltpu.SemaphoreType.DMA,
            pltpu.SemaphoreType.DMA,
        ],
    )
    def k(pq_hbm, dst_hbm, src_hbm, g_hbm, db0, sb0, db1, sb1, pi0, qi0,
          pi1, qi1, pb0, qb0, pb1, qb1, s0, s1, s2, s3, si0, si1):
        wid = lax.axis_index("s") * 2 + lax.axis_index("c")
        nch = (nch_total - wid + _NW - 1) // _NW

        def load_idx(i, dbuf, sbuf, sem):
            c = wid + i * _NW
            off = pl.multiple_of(c * _GC, 8)
            cpd = pltpu.async_copy(dst_hbm.at[pl.ds(off, _GC)], dbuf, sem)
            cps = pltpu.async_copy(src_hbm.at[pl.ds(off, _GC)], sbuf, sem)
            return (cpd, cps)

        def issue(cpi, dbuf, sbuf, pidx, qidx, pbuf, qbuf, semp, semq):
            cpi[0].wait()
            cpi[1].wait()
            for kk in range(_GC // 16):
                d = dbuf[pl.ds(kk * 16, 16)]
                s = sbuf[pl.ds(kk * 16, 16)]
                pidx[pl.ds(kk * 16, 16)] = d + d
                qidx[pl.ds(kk * 16, 16)] = s + s + 1
            return (pltpu.async_copy(pq_hbm.at[pidx], pbuf, semp),
                    pltpu.async_copy(pq_hbm.at[qidx], qbuf, semq))

        def finish(i, cps, pbuf, qbuf):
            c = wid + i * _NW
            off = pl.multiple_of(c * _GC, 8)
            cps[0].wait()
            cps[1].wait()

            def add(j, _):
                for f in range(16):
                    qbuf[j, pl.ds(f * 16, 16)] = (
                        qbuf[j, pl.ds(f * 16, 16)] + pbuf[j, pl.ds(f * 16, 16)])
                return 0

            lax.fori_loop(0, _GC, add, 0)
            pltpu.sync_copy(qbuf, g_hbm.at[pl.ds(off, _GC)])

        def pair(c2, _):
            a = c2 * 2
            cia = load_idx(a, db0, sb0, si0)
            cib = load_idx(a + 1, db1, sb1, si1)
            cpa = issue(cia, db0, sb0, pi0, qi0, pb0, qb0, s0, s1)
            cpb = issue(cib, db1, sb1, pi1, qi1, pb1, qb1, s2, s3)
            finish(a, cpa, pb0, qb0)
            finish(a + 1, cpb, pb1, qb1)
            return 0

        lax.fori_loop(0, nch >> 1, pair, 0)

        def tail(_):
            cia = load_idx(nch - 1, db0, sb0, si0)
            cpa = issue(cia, db0, sb0, pi0, qi0, pb0, qb0, s0, s1)
            finish(nch - 1, cpa, pb0, qb0)
            return 0

        lax.cond(nch & 1, tail, lambda _: 0, 0)

    return k(pq.reshape(-1, 256), dst, src)


def _scatter_max(h2, le, lr, cnt):
    @functools.partial(
        pl.kernel, mesh=_sc_mesh(),
        compiler_params=pltpu.CompilerParams(needs_layout_passes=False),
        out_type=jax.ShapeDtypeStruct((_NPAD * 128,), jnp.float32),
        scratch_types=[
            pltpu.VMEM(((_NPW + 1) * 128,), jnp.float32),
            pltpu.VMEM((128,), jnp.int32),
            pltpu.VMEM((128,), jnp.int32),
            pltpu.VMEM((144,), jnp.int32),
            pltpu.VMEM((144,), jnp.int32),
            pltpu.VMEM((128, 128), jnp.float32),
            pltpu.VMEM((128, 128), jnp.float32),
            pltpu.VMEM((16,), jnp.int32),
            pltpu.SemaphoreType.DMA,
            pltpu.SemaphoreType.DMA,
        ],
    )
    def k(h2_hbm, le_hbm, lr_hbm, cnt_hbm, out_hbm, acc, ev0, ev1, rv0, rv1,
          hr0, hr1, cv, sem0, sem1):
        wid = lax.axis_index("s") * 2 + lax.axis_index("c")
        lbase = wid * _LCAP
        zeros = jnp.zeros((16,), jnp.float32)

        def init(i, _):
            acc[pl.ds(i * 16, 16)] = zeros
            return 0

        lax.fori_loop(0, (_NPW + 1) * 8, init, 0)

        pltpu.sync_copy(cnt_hbm.at[pl.ds(pl.multiple_of(wid * 16, 8), 16)], cv)
        nch = cv[...][0] >> 7

        def load(c, ev, rv, hr, sem):
            pltpu.sync_copy(le_hbm.at[pl.ds(pl.multiple_of(lbase + c * 128, 8), 128)], ev)
            pltpu.sync_copy(lr_hbm.at[pl.ds(pl.multiple_of(lbase + c * 128, 8), 128)],
                            rv.at[pl.ds(0, 128)])
            return pltpu.async_copy(h2_hbm.at[ev], hr, sem)

        def proc(rv, hr):
            def edge(j, rb):
                rb_next = rv[pl.ds(j + 1, 16)][0]
                for f in range(8):
                    off = rb + f * 16
                    acc[pl.ds(off, 16)] = jnp.maximum(
                        acc[pl.ds(off, 16)], hr[j, pl.ds(f * 16, 16)])
                return rb_next

            lax.fori_loop(0, 128, edge, rv[pl.ds(0, 16)][0])

        def pair(c2, _):
            a = c2 * 2
            cpa = load(a, ev0, rv0, hr0, sem0)
            cpb = load(a + 1, ev1, rv1, hr1, sem1)
            cpa.wait()
            proc(rv0, hr0)
            cpb.wait()
            proc(rv1, hr1)
            return 0

        lax.fori_loop(0, nch >> 1, pair, 0)

        def tail(_):
            load(nch - 1, ev0, rv0, hr0, sem0).wait()
            proc(rv0, hr0)
            return 0

        lax.cond(nch & 1, tail, lambda _: 0, 0)
        pltpu.sync_copy(acc.at[pl.ds(0, _NPW * 128)],
                        out_hbm.at[pl.ds(pl.multiple_of(wid * _NPW * 128, 8), _NPW * 128)])

    return k(h2, le, lr, cnt)


def _mm_body(x_ref, w_ref, b_ref, o_ref, *, relu_out):
    acc = jnp.dot(x_ref[...], w_ref[...], preferred_element_type=jnp.float32)
    acc = acc + b_ref[...]
    if relu_out:
        acc = jnp.maximum(acc, 0.0)
    o_ref[...] = acc


def _matmul_bias(x, w, b, block_rows, relu_out=False):
    n, k = x.shape
    ko, m = w.shape
    assert k == ko and n % block_rows == 0
    grid = (n // block_rows,)
    return pl.pallas_call(
        functools.partial(_mm_body, relu_out=relu_out),
        grid=grid,
        in_specs=[
            pl.BlockSpec((block_rows, k), lambda i: (i, 0)),
            pl.BlockSpec((k, m), lambda i: (0, 0)),
            pl.BlockSpec((1, m), lambda i: (0, 0)),
        ],
        out_specs=pl.BlockSpec((block_rows, m), lambda i: (i, 0)),
        out_shape=jax.ShapeDtypeStruct((n, m), jnp.float32),
    )(x, w, b.reshape(1, m))


def _edge_mlp_body(g_ref, w_ref, b_ref, o_ref):
    g = jnp.maximum(g_ref[...], 0.0)
    acc = jnp.dot(g, w_ref[...], preferred_element_type=jnp.float32)
    o_ref[...] = jnp.maximum(acc + b_ref[...], 0.0)


def _edge_mlp(g, w2, b2, block_rows=2000):
    e, k = g.shape
    _, m = w2.shape
    grid = (e // block_rows,)
    return pl.pallas_call(
        _edge_mlp_body,
        grid=grid,
        in_specs=[
            pl.BlockSpec((block_rows, k), lambda i: (i, 0)),
            pl.BlockSpec((k, m), lambda i: (0, 0)),
            pl.BlockSpec((1, m), lambda i: (0, 0)),
        ],
        out_specs=pl.BlockSpec((block_rows, m), lambda i: (i, 0)),
        out_shape=jax.ShapeDtypeStruct((e, m), jnp.float32),
    )(g, w2, b2.reshape(1, m))


def _gru2_body(x_ref, h0_ref, h1_ref, wi0_ref, wh0_ref, wi1_ref, wh1_ref,
               bi0_ref, bh0_ref, bi1_ref, bh1_ref, h0o_ref, h1o_ref):
    H = 128

    def cell(x, h, wi, wh, bi, bh):
        gi = jnp.dot(x, wi, preferred_element_type=jnp.float32) + bi
        gh = jnp.dot(h, wh, preferred_element_type=jnp.float32) + bh
        r = jax.nn.sigmoid(gi[:, :H] + gh[:, :H])
        zg = jax.nn.sigmoid(gi[:, H:2 * H] + gh[:, H:2 * H])
        ng = jnp.tanh(gi[:, 2 * H:] + r * gh[:, 2 * H:])
        return (1.0 - zg) * ng + zg * h

    h0 = cell(x_ref[...], h0_ref[...], wi0_ref[...], wh0_ref[...],
              bi0_ref[...], bh0_ref[...])
    h1 = cell(h0, h1_ref[...], wi1_ref[...], wh1_ref[...],
              bi1_ref[...], bh1_ref[...])
    h0o_ref[...] = h0
    h1o_ref[...] = h1


def _gru2(xpad, h0, h1, wi0t, wh0t, wi1t, wh1t, bi0, bh0, bi1, bh1,
          block_rows=2000):
    n, kx = xpad.shape
    H = 128
    grid = (n // block_rows,)
    out_sd = jax.ShapeDtypeStruct((n, H), jnp.float32)
    return pl.pallas_call(
        _gru2_body,
        grid=grid,
        in_specs=[
            pl.BlockSpec((block_rows, kx), lambda i: (i, 0)),
            pl.BlockSpec((block_rows, H), lambda i: (i, 0)),
            pl.BlockSpec((block_rows, H), lambda i: (i, 0)),
            pl.BlockSpec((kx, 3 * H), lambda i: (0, 0)),
            pl.BlockSpec((H, 3 * H), lambda i: (0, 0)),
            pl.BlockSpec((H, 3 * H), lambda i: (0, 0)),
            pl.BlockSpec((H, 3 * H), lambda i: (0, 0)),
            pl.BlockSpec((1, 3 * H), lambda i: (0, 0)),
            pl.BlockSpec((1, 3 * H), lambda i: (0, 0)),
            pl.BlockSpec((1, 3 * H), lambda i: (0, 0)),
            pl.BlockSpec((1, 3 * H), lambda i: (0, 0)),
        ],
        out_specs=[
            pl.BlockSpec((block_rows, H), lambda i: (i, 0)),
            pl.BlockSpec((block_rows, H), lambda i: (i, 0)),
        ],
        out_shape=[out_sd, out_sd],
    )(xpad, h0, h1, wi0t, wh0t, wi1t, wh1t,
      bi0.reshape(1, -1), bh0.reshape(1, -1),
      bi1.reshape(1, -1), bh1.reshape(1, -1))


def _edge_conv_fast(x, src, dst, w1, b1, w2, b2, n, lists):
    d = x.shape[1]
    w1a, w1b = w1[:d], w1[d:]
    wcat = jnp.concatenate([w1a - w1b, w1b], axis=1)
    bcat = jnp.concatenate([b1, jnp.zeros_like(b1)])
    pq = _matmul_bias(x, wcat, bcat, block_rows=2000)
    g = _gather_add(pq, dst, src)
    h2 = _edge_mlp(g, w2, b2)
    le, lr, cnts = lists
    out_flat = _scatter_max(h2, le, lr, cnts)
    return out_flat.reshape(_NPAD, 128)[:n]


def kernel(data, edge_index, z, t, a, h, We1, be1, We2, be2, Wi0, Wh0, bi0,
           bh0, Wi1, Wh1, bi1, bh1, Wd1, bd1, Wd2, bd2):
    n = data.shape[0]
    src = edge_index[0]
    dst = edge_index[1]

    lists = _build_lists(dst)
    enc = _edge_conv_fast(data, src, dst, We1, be1, We2, be2, n, lists)

    x = jnp.concatenate([z, enc, t, a], axis=-1)
    xpad = jnp.pad(x, ((0, 0), (0, 256 - x.shape[1])))
    wi0t = jnp.pad(Wi0.T, ((0, 256 - Wi0.shape[1]), (0, 0)))
    h0, h1 = _gru2(xpad, h[0], h[1], wi0t, Wh0.T, Wi1.T, Wh1.T,
                   bi0, bh0, bi1, bh1)
    h_new = jnp.stack([h0, h1], axis=0)

    out = _edge_conv_fast(h1, src, dst, Wd1, bd1, Wd2, bd2, n, lists)
    return (h_new, out)


# speedup vs baseline: 1.0356x; 1.0356x over previous
import functools

import jax
import jax.numpy as jnp
from jax import lax
from jax.experimental import pallas as pl
from jax.experimental.pallas import tpu as pltpu
from jax.experimental.pallas import tpu_sc as plsc

_NW = 32
_NPW = 320
_NPAD = _NW * _NPW
_SINK = _NPW
_LCAP = 43 * 4096


def _sc_mesh():
    return plsc.VectorSubcoreMesh(core_axis_name="c", subcore_axis_name="s")


def _build_lists(dst):
    e_total = dst.shape[0]
    assert e_total % 4000 == 0
    nblocks = e_total // 4000

    @functools.partial(
        pl.kernel, mesh=_sc_mesh(),
        compiler_params=pltpu.CompilerParams(needs_layout_passes=False),
        out_type=[
            jax.ShapeDtypeStruct((_NW * _LCAP,), jnp.int32),
            jax.ShapeDtypeStruct((_NW * _LCAP,), jnp.int32),
            jax.ShapeDtypeStruct((_NW * 16,), jnp.int32),
        ],
        scratch_types=[
            pltpu.VMEM((4000,), jnp.int32),
            pltpu.VMEM((8224,), jnp.int32),
            pltpu.VMEM((8224,), jnp.int32),
            pltpu.VMEM((16,), jnp.int32),
            pltpu.SemaphoreType.DMA,
        ],
    )
    def k(dst_hbm, le_hbm, lr_hbm, cnt_hbm, dbuf, be, br, cv, sem):
        wid = lax.axis_index("s") * 2 + lax.axis_index("c")
        lbase = wid * _LCAP
        lo = wid * _NPW
        lane = lax.iota(jnp.int32, 16)

        def block(b, carry):
            cnt, hcnt = carry
            pltpu.sync_copy(dst_hbm.at[pl.ds(pl.multiple_of(b * 4000, 8), 4000)], dbuf)

            def scan16(k2, cnt):
                d = dbuf[pl.ds(k2 * 16, 16)]
                m = (d >= lo) & (d < lo + _NPW)
                eid = b * 4000 + k2 * 16 + lane
                mi = m.astype(jnp.int32)
                pos = plsc.cumsum(mi)
                tgt = cnt + pos - mi
                plsc.store_scatter(be, [tgt], eid, mask=m)
                plsc.store_scatter(br, [tgt], (d - lo) * 128, mask=m)
                return cnt + pos[15]

            cnt = lax.fori_loop(0, 250, scan16, cnt)

            def flush(args):
                cnt, hcnt = args
                pltpu.sync_copy(be.at[pl.ds(0, 4096)],
                                le_hbm.at[pl.ds(pl.multiple_of(lbase + hcnt, 8), 4096)])
                pltpu.sync_copy(br.at[pl.ds(0, 4096)],
                                lr_hbm.at[pl.ds(pl.multiple_of(lbase + hcnt, 8), 4096)])

                def mv(i, _):
                    be[pl.ds(i * 16, 16)] = be[pl.ds(4096 + i * 16, 16)]
                    br[pl.ds(i * 16, 16)] = br[pl.ds(4096 + i * 16, 16)]
                    return 0

                lax.fori_loop(0, 256, mv, 0)
                return cnt - 4096, hcnt + 4096

            return lax.cond(cnt >= 4096, flush, lambda a: a, (cnt, hcnt))

        cnt, hcnt = lax.fori_loop(0, nblocks, block,
                                  (jnp.int32(0), jnp.int32(0)))
        for i in range(8):
            be[pl.ds(cnt + i * 16, 16)] = jnp.zeros((16,), jnp.int32)
            br[pl.ds(cnt + i * 16, 16)] = jnp.full((16,), _SINK * 128, jnp.int32)
        cntp = ((cnt + 127) >> 7) << 7

        def fflush(i, _):
            pltpu.sync_copy(be.at[pl.ds(i * 4096, 4096)],
                            le_hbm.at[pl.ds(pl.multiple_of(lbase + hcnt + i * 4096, 8), 4096)])
            pltpu.sync_copy(br.at[pl.ds(i * 4096, 4096)],
                            lr_hbm.at[pl.ds(pl.multiple_of(lbase + hcnt + i * 4096, 8), 4096)])
            return 0

        lax.fori_loop(0, (cntp + 4095) >> 12, fflush, 0)
        cv[...] = jnp.full((16,), hcnt + cntp, jnp.int32)
        pltpu.sync_copy(cv, cnt_hbm.at[pl.ds(pl.multiple_of(wid * 16, 8), 16)])

    return k(dst)


_GC = 64


def _gather_add(pq, dst, src):
    e_total = dst.shape[0]
    assert e_total % _GC == 0
    nch_total = e_total // _GC

    @functools.partial(
        pl.kernel, mesh=_sc_mesh(),
        compiler_params=pltpu.CompilerParams(needs_layout_passes=False),
        out_type=jax.ShapeDtypeStruct((e_total, 256), jnp.float32),
        scratch_types=[
            pltpu.VMEM((_GC,), jnp.int32),
            pltpu.VMEM((_GC,), jnp.int32),
            pltpu.VMEM((_GC,), jnp.int32),
            pltpu.VMEM((_GC,), jnp.int32),
            pltpu.VMEM((_GC,), jnp.int32),
            pltpu.VMEM((_GC,), jnp.int32),
            pltpu.VMEM((_GC, 256), jnp.float32),
            pltpu.VMEM((_GC, 256), jnp.float32),
            pltpu.VMEM((_GC, 256), jnp.float32),
            pltpu.VMEM((_GC, 256), jnp.float32),
            pltpu.SemaphoreType.DMA,
            pltpu.SemaphoreType.DMA,
            pltpu.SemaphoreType.DMA,
            pltpu.SemaphoreType.DMA,
        ],
    )
    def k(pq_hbm, dst_hbm, src_hbm, g_hbm, dbuf, sbuf, pi0, qi0, pi1, qi1,
          pb0, qb0, pb1, qb1, s0, s1, s2, s3):
        wid = lax.axis_index("s") * 2 + lax.axis_index("c")
        nch = (nch_total - wid + _NW - 1) // _NW

        def issue(i, pidx, qidx, pbuf, qbuf, semp, semq):
            c = wid + i * _NW
            off = pl.multiple_of(c * _GC, 8)
            pltpu.sync_copy(dst_hbm.at[pl.ds(off, _GC)], dbuf)
            pltpu.sync_copy(src_hbm.at[pl.ds(off, _GC)], sbuf)
            for kk in range(_GC // 16):
                d = dbuf[pl.ds(kk * 16, 16)]
                s = sbuf[pl.ds(kk * 16, 16)]
                pidx[pl.ds(kk * 16, 16)] = d + d
                qidx[pl.ds(kk * 16, 16)] = s + s + 1
            return (pltpu.async_copy(pq_hbm.at[pidx], pbuf, semp),
                    pltpu.async_copy(pq_hbm.at[qidx], qbuf, semq))

        def finish(i, cps, pbuf, qbuf):
            c = wid + i * _NW
            off = pl.multiple_of(c * _GC, 8)
            cps[0].wait()
            cps[1].wait()

            def add(j, _):
                for f in range(16):
                    qbuf[j, pl.ds(f * 16, 16)] = (
                        qbuf[j, pl.ds(f * 16, 16)] + pbuf[j, pl.ds(f * 16, 16)])
                return 0

            lax.fori_loop(0, _GC, add, 0)
            pltpu.sync_copy(qbuf, g_hbm.at[pl.ds(off, _GC)])

        def pair(c2, _):
            a = c2 * 2
            cpa = issue(a, pi0, qi0, pb0, qb0, s0, s1)
            cpb = issue(a + 1, pi1, qi1, pb1, qb1, s2, s3)
            finish(a, cpa, pb0, qb0)
            finish(a + 1, cpb, pb1, qb1)
            return 0

        lax.fori_loop(0, nch >> 1, pair, 0)

        def tail(_):
            cpa = issue(nch - 1, pi0, qi0, pb0, qb0, s0, s1)
            finish(nch - 1, cpa, pb0, qb0)
            return 0

        lax.cond(nch & 1, tail, lambda _: 0, 0)

    return k(pq.reshape(-1, 256), dst, src)


def _scatter_max(h2, le, lr, cnt):
    @functools.partial(
        pl.kernel, mesh=_sc_mesh(),
        compiler_params=pltpu.CompilerParams(needs_layout_passes=False),
        out_type=jax.ShapeDtypeStruct((_NPAD * 128,), jnp.float32),
        scratch_types=[
            pltpu.VMEM(((_NPW + 1) * 128,), jnp.float32),
            pltpu.VMEM((128,), jnp.int32),
            pltpu.VMEM((128,), jnp.int32),
            pltpu.VMEM((144,), jnp.int32),
            pltpu.VMEM((144,), jnp.int32),
            pltpu.VMEM((128, 128), jnp.float32),
            pltpu.VMEM((128, 128), jnp.float32),
            pltpu.VMEM((16,), jnp.int32),
            pltpu.SemaphoreType.DMA,
            pltpu.SemaphoreType.DMA,
        ],
    )
    def k(h2_hbm, le_hbm, lr_hbm, cnt_hbm, out_hbm, acc, ev0, ev1, rv0, rv1,
          hr0, hr1, cv, sem0, sem1):
        wid = lax.axis_index("s") * 2 + lax.axis_index("c")
        lbase = wid * _LCAP
        zeros = jnp.zeros((16,), jnp.float32)

        def init(i, _):
            acc[pl.ds(i * 16, 16)] = zeros
            return 0

        lax.fori_loop(0, (_NPW + 1) * 8, init, 0)

        pltpu.sync_copy(cnt_hbm.at[pl.ds(pl.multiple_of(wid * 16, 8), 16)], cv)
        nch = cv[...][0] >> 7

        def load(c, ev, rv, hr, sem):
            pltpu.sync_copy(le_hbm.at[pl.ds(pl.multiple_of(lbase + c * 128, 8), 128)], ev)
            pltpu.sync_copy(lr_hbm.at[pl.ds(pl.multiple_of(lbase + c * 128, 8), 128)],
                            rv.at[pl.ds(0, 128)])
            return pltpu.async_copy(h2_hbm.at[ev], hr, sem)

        def proc(rv, hr):
            def edge(j, rb):
                rb_next = rv[pl.ds(j + 1, 16)][0]
                for f in range(8):
                    off = rb + f * 16
                    acc[pl.ds(off, 16)] = jnp.maximum(
                        acc[pl.ds(off, 16)], hr[j, pl.ds(f * 16, 16)])
                return rb_next

            lax.fori_loop(0, 128, edge, rv[pl.ds(0, 16)][0])

        def pair(c2, _):
            a = c2 * 2
            cpa = load(a, ev0, rv0, hr0, sem0)
            cpb = load(a + 1, ev1, rv1, hr1, sem1)
            cpa.wait()
            proc(rv0, hr0)
            cpb.wait()
            proc(rv1, hr1)
            return 0

        lax.fori_loop(0, nch >> 1, pair, 0)

        def tail(_):
            load(nch - 1, ev0, rv0, hr0, sem0).wait()
            proc(rv0, hr0)
            return 0

        lax.cond(nch & 1, tail, lambda _: 0, 0)
        pltpu.sync_copy(acc.at[pl.ds(0, _NPW * 128)],
                        out_hbm.at[pl.ds(pl.multiple_of(wid * _NPW * 128, 8), _NPW * 128)])

    return k(h2, le, lr, cnt)


def _mm_body(x_ref, w_ref, b_ref, o_ref, *, relu_out):
    acc = jnp.dot(x_ref[...], w_ref[...], preferred_element_type=jnp.float32)
    acc = acc + b_ref[...]
    if relu_out:
        acc = jnp.maximum(acc, 0.0)
    o_ref[...] = acc


def _matmul_bias(x, w, b, block_rows, relu_out=False):
    n, k = x.shape
    ko, m = w.shape
    assert k == ko and n % block_rows == 0
    grid = (n // block_rows,)
    return pl.pallas_call(
        functools.partial(_mm_body, relu_out=relu_out),
        grid=grid,
        in_specs=[
            pl.BlockSpec((block_rows, k), lambda i: (i, 0)),
            pl.BlockSpec((k, m), lambda i: (0, 0)),
            pl.BlockSpec((1, m), lambda i: (0, 0)),
        ],
        out_specs=pl.BlockSpec((block_rows, m), lambda i: (i, 0)),
        out_shape=jax.ShapeDtypeStruct((n, m), jnp.float32),
    )(x, w, b.reshape(1, m))


def _edge_mlp_body(g_ref, w_ref, b_ref, o_ref):
    g = jnp.maximum(g_ref[...], 0.0)
    acc = jnp.dot(g, w_ref[...], preferred_element_type=jnp.float32)
    o_ref[...] = jnp.maximum(acc + b_ref[...], 0.0)


def _edge_mlp(g, w2, b2, block_rows=2000):
    e, k = g.shape
    _, m = w2.shape
    grid = (e // block_rows,)
    return pl.pallas_call(
        _edge_mlp_body,
        grid=grid,
        in_specs=[
            pl.BlockSpec((block_rows, k), lambda i: (i, 0)),
            pl.BlockSpec((k, m), lambda i: (0, 0)),
            pl.BlockSpec((1, m), lambda i: (0, 0)),
        ],
        out_specs=pl.BlockSpec((block_rows, m), lambda i: (i, 0)),
        out_shape=jax.ShapeDtypeStruct((e, m), jnp.float32),
    )(g, w2, b2.reshape(1, m))


def _gru2_body(x_ref, h0_ref, h1_ref, wi0_ref, wh0_ref, wi1_ref, wh1_ref,
               bi0_ref, bh0_ref, bi1_ref, bh1_ref, h0o_ref, h1o_ref):
    H = 128

    def cell(x, h, wi, wh, bi, bh):
        gi = jnp.dot(x, wi, preferred_element_type=jnp.float32) + bi
        gh = jnp.dot(h, wh, preferred_element_type=jnp.float32) + bh
        r = jax.nn.sigmoid(gi[:, :H] + gh[:, :H])
        zg = jax.nn.sigmoid(gi[:, H:2 * H] + gh[:, H:2 * H])
        ng = jnp.tanh(gi[:, 2 * H:] + r * gh[:, 2 * H:])
        return (1.0 - zg) * ng + zg * h

    h0 = cell(x_ref[...], h0_ref[...], wi0_ref[...], wh0_ref[...],
              bi0_ref[...], bh0_ref[...])
    h1 = cell(h0, h1_ref[...], wi1_ref[...], wh1_ref[...],
              bi1_ref[...], bh1_ref[...])
    h0o_ref[...] = h0
    h1o_ref[...] = h1


def _gru2(xpad, h0, h1, wi0t, wh0t, wi1t, wh1t, bi0, bh0, bi1, bh1,
          block_rows=2000):
    n, kx = xpad.shape
    H = 128
    grid = (n // block_rows,)
    out_sd = jax.ShapeDtypeStruct((n, H), jnp.float32)
    return pl.pallas_call(
        _gru2_body,
        grid=grid,
        in_specs=[
            pl.BlockSpec((block_rows, kx), lambda i: (i, 0)),
            pl.BlockSpec((block_rows, H), lambda i: (i, 0)),
            pl.BlockSpec((block_rows, H), lambda i: (i, 0)),
            pl.BlockSpec((kx, 3 * H), lambda i: (0, 0)),
            pl.BlockSpec((H, 3 * H), lambda i: (0, 0)),
            pl.BlockSpec((H, 3 * H), lambda i: (0, 0)),
            pl.BlockSpec((H, 3 * H), lambda i: (0, 0)),
            pl.BlockSpec((1, 3 * H), lambda i: (0, 0)),
            pl.BlockSpec((1, 3 * H), lambda i: (0, 0)),
            pl.BlockSpec((1, 3 * H), lambda i: (0, 0)),
            pl.BlockSpec((1, 3 * H), lambda i: (0, 0)),
        ],
        out_specs=[
            pl.BlockSpec((block_rows, H), lambda i: (i, 0)),
            pl.BlockSpec((block_rows, H), lambda i: (i, 0)),
        ],
        out_shape=[out_sd, out_sd],
    )(xpad, h0, h1, wi0t, wh0t, wi1t, wh1t,
      bi0.reshape(1, -1), bh0.reshape(1, -1),
      bi1.reshape(1, -1), bh1.reshape(1, -1))


def _edge_conv_fast(x, src, dst, w1, b1, w2, b2, n, lists):
    d = x.shape[1]
    w1a, w1b = w1[:d], w1[d:]
    wcat = jnp.concatenate([w1a - w1b, w1b], axis=1)
    bcat = jnp.concatenate([b1, jnp.zeros_like(b1)])
    pq = _matmul_bias(x, wcat, bcat, block_rows=2000)
    g = _gather_add(pq, dst, src)
    h2 = _edge_mlp(g, w2, b2)
    le, lr, cnts = lists
    out_flat = _scatter_max(h2, le, lr, cnts)
    return out_flat.reshape(_NPAD, 128)[:n]


def kernel(data, edge_index, z, t, a, h, We1, be1, We2, be2, Wi0, Wh0, bi0,
           bh0, Wi1, Wh1, bi1, bh1, Wd1, bd1, Wd2, bd2):
    n = data.shape[0]
    src = edge_index[0]
    dst = edge_index[1]

    lists = _build_lists(dst)
    enc = _edge_conv_fast(data, src, dst, We1, be1, We2, be2, n, lists)

    x = jnp.concatenate([z, enc, t, a], axis=-1)
    xpad = jnp.pad(x, ((0, 0), (0, 256 - x.shape[1])))
    wi0t = jnp.pad(Wi0.T, ((0, 256 - Wi0.shape[1]), (0, 0)))
    h0, h1 = _gru2(xpad, h[0], h[1], wi0t, Wh0.T, Wi1.T, Wh1.T,
                   bi0, bh0, bi1, bh1)
    h_new = jnp.stack([h0, h1], axis=0)

    out = _edge_conv_fast(h1, src, dst, Wd1, bd1, Wd2, bd2, n, lists)
    return (h_new, out)
